# baseline (speedup 1.0000x reference)
import jax
import jax.numpy as jnp
from jax import lax
from jax.experimental import pallas as pl
from jax.experimental.pallas import tpu as pltpu
from jax.experimental.pallas import tpu_sc as plsc

N = 10000
E = 160000
D = 256
H = 128
NS = 16
L = 16

EPT = E // NS
EC = 80
NCH = EPT // EC
RC = 80
NRCH = N // RC
RPT = -(-NRCH // NS)
NP = 10240


def _sc_body(xview, src2, dst, h0o, h1o, dego, acc, idx0, idx1, idx2,
             dst0, dst1, dst2, rows0, rows1, degloc,
             sg0, sg1, ss0, ss1, si0, si1, si2):
    c = lax.axis_index("c")
    s = lax.axis_index("s")
    zvec = jnp.zeros((L,), dtype=jnp.float32)
    ones = jnp.ones((L,), dtype=jnp.float32)
    idx_b = (idx0, idx1, idx2)
    dst_b = (dst0, dst1, dst2)
    rows_b = (rows0, rows1)
    sg_b = (sg0, sg1)
    ss_b = (ss0, ss1)
    si_b = (si0, si1, si2)

    def zrow(r, carry):
        for j in range(H // L):
            rows0[r, pl.ds(j * L, L)] = zvec
        return carry

    lax.fori_loop(0, RC, zrow, 0)

    def zdeg(i, carry):
        degloc[pl.ds(i * L, L)] = zvec
        return carry

    lax.fori_loop(0, N // L, zdeg, 0)

    for k in range(RPT):
        cid = k * NS + s

        @pl.when(cid < NRCH)
        def _():
            pltpu.sync_copy(rows0, acc.at[pl.ds(cid * RC, RC)])

    plsc.subcore_barrier()

    base = s * EPT

    sbase = c * E + base

    def start_load_idx(k, a):
        eoff = k * EC
        pltpu.async_copy(src2.at[pl.ds(sbase + eoff, EC)], idx_b[a], si_b[a])
        pltpu.async_copy(dst.at[pl.ds(base + eoff, EC)], dst_b[a], si_b[a])

    def wait_load_idx(k, a):
        eoff = k * EC
        pltpu.make_async_copy(src2.at[pl.ds(sbase + eoff, EC)], idx_b[a],
                              si_b[a]).wait()
        pltpu.make_async_copy(dst.at[pl.ds(base + eoff, EC)], dst_b[a],
                              si_b[a]).wait()

    def start_gather(a, b):
        pltpu.async_copy(xview.at[idx_b[a]], rows_b[b], sg_b[b])

    def wait_gather(a, b):
        pltpu.make_async_copy(xview.at[idx_b[a]], rows_b[b], sg_b[b]).wait()

    def start_scatter(a, b):
        pltpu.async_copy(rows_b[b], acc.at[dst_b[a]], ss_b[b], add=True)

    def wait_scatter(a, b):
        pltpu.make_async_copy(rows_b[b], acc.at[dst_b[a]], ss_b[b]).wait()

    start_load_idx(0, 0)
    wait_load_idx(0, 0)
    start_gather(0, 0)
    start_load_idx(1, 1)
    start_load_idx(2, 2)

    def body(k6, carry):
        for u in range(6):
            k = k6 * 6 + u
            a = u % 3
            b = u % 2

            @pl.when(k < NCH)
            def _():
                wait_gather(a, b)

                @pl.when(k >= 1)
                def _():
                    wait_scatter((a + 2) % 3, 1 - b)

                @pl.when(k + 1 < NCH)
                def _():
                    wait_load_idx(k + 1, (a + 1) % 3)
                    start_gather((a + 1) % 3, 1 - b)

                start_scatter(a, b)

                @pl.when(c == 0)
                def _():
                    for j in range(EC // L):
                        iv = dst_b[a][pl.ds(j * L, L)]
                        plsc.addupdate_scatter(degloc, [iv], ones)

                @pl.when(k + 3 < NCH)
                def _():
                    start_load_idx(k + 3, a)

        return carry

    lax.fori_loop(0, (NCH + 5) // 6, body, 0)
    wait_scatter((NCH - 1) % 3, (NCH - 1) % 2)

    @pl.when(c == 0)
    def _():
        pltpu.sync_copy(degloc, dego.at[pl.ds(s * NP, N)])

    plsc.subcore_barrier()

    for k in range(RPT):
        cid = k * NS + s

        @pl.when(cid < NRCH)
        def _():
            row0 = cid * RC

            @pl.when(c == 0)
            def _():
                pltpu.sync_copy(acc.at[pl.ds(row0, RC)],
                                h0o.at[pl.ds(row0, RC)])

            @pl.when(c == 1)
            def _():
                pltpu.sync_copy(acc.at[pl.ds(row0, RC)],
                                h1o.at[pl.ds(row0, RC)])


_sc_agg = pl.kernel(
    _sc_body,
    out_type=(
        jax.ShapeDtypeStruct((N, H), jnp.float32),
        jax.ShapeDtypeStruct((N, H), jnp.float32),
        jax.ShapeDtypeStruct((NS * NP,), jnp.float32),
    ),
    mesh=plsc.VectorSubcoreMesh(core_axis_name="c", subcore_axis_name="s"),
    compiler_params=pltpu.CompilerParams(needs_layout_passes=False),
    scratch_types=[
        pltpu.VMEM_SHARED((N, H), jnp.float32),
        pltpu.VMEM((EC,), jnp.int32),
        pltpu.VMEM((EC,), jnp.int32),
        pltpu.VMEM((EC,), jnp.int32),
        pltpu.VMEM((EC,), jnp.int32),
        pltpu.VMEM((EC,), jnp.int32),
        pltpu.VMEM((EC,), jnp.int32),
        pltpu.VMEM((EC, H), jnp.float32),
        pltpu.VMEM((EC, H), jnp.float32),
        pltpu.VMEM((N,), jnp.float32),
        pltpu.SemaphoreType.DMA,
        pltpu.SemaphoreType.DMA,
        pltpu.SemaphoreType.DMA,
        pltpu.SemaphoreType.DMA,
        pltpu.SemaphoreType.DMA,
        pltpu.SemaphoreType.DMA,
        pltpu.SemaphoreType.DMA,
    ],
)


BN = 2000


def _tc_body(x_ref, h0_ref, h1_ref, dg_ref, ws_ref, wn0_ref, wn1_ref,
             b_ref, o_ref):
    deg = jnp.sum(dg_ref[...], axis=1)
    rdeg = (1.0 / jnp.maximum(deg, 1.0))[:, None]
    o_ref[...] = (
        jnp.dot(x_ref[...], ws_ref[...], preferred_element_type=jnp.float32)
        + jnp.dot(h0_ref[...] * rdeg, wn0_ref[...],
                  preferred_element_type=jnp.float32)
        + jnp.dot(h1_ref[...] * rdeg, wn1_ref[...],
                  preferred_element_type=jnp.float32)
        + b_ref[...]
    )


_tc_dense = pl.pallas_call(
    _tc_body,
    grid=(N // BN,),
    in_specs=[
        pl.BlockSpec((BN, D), lambda i: (i, 0)),
        pl.BlockSpec((BN, H), lambda i: (i, 0)),
        pl.BlockSpec((BN, H), lambda i: (i, 0)),
        pl.BlockSpec((BN, NS), lambda i: (i, 0)),
        pl.BlockSpec((D, D), lambda i: (0, 0)),
        pl.BlockSpec((H, D), lambda i: (0, 0)),
        pl.BlockSpec((H, D), lambda i: (0, 0)),
        pl.BlockSpec((1, D), lambda i: (0, 0)),
    ],
    out_specs=pl.BlockSpec((BN, D), lambda i: (i, 0)),
    out_shape=jax.ShapeDtypeStruct((N, D), jnp.float32),
)


def kernel(x, edge_index, W_self, W_neigh, b):
    src = edge_index[0].astype(jnp.int32)
    dst = edge_index[1].astype(jnp.int32)
    xview = x.reshape(2 * N, H)
    s2 = src * 2
    src2 = jnp.concatenate([s2, s2 + 1])
    h0, h1, dego = _sc_agg(xview, src2, dst)
    deg16 = dego.reshape(NS, NP)[:, :N].T
    return _tc_dense(x, h0, h1, deg16, W_self, W_neigh[:H], W_neigh[H:],
                     b.reshape(1, D))


# speedup vs baseline: 1.0017x; 1.0017x over previous
import jax
import jax.numpy as jnp
from jax import lax
from jax.experimental import pallas as pl
from jax.experimental.pallas import tpu as pltpu
from jax.experimental.pallas import tpu_sc as plsc

N = 10000
E = 160000
D = 256
H = 128
NS = 16
L = 16

EPT = E // NS
EC = 80
NCH = EPT // EC
RC = 80
NRCH = N // RC
RPT = -(-NRCH // NS)
NP = 10240


def _sc_body(xview, src2, dst, h0o, h1o, dego, acc, idx0, idx1, dst0, dst1,
             rows0, rows1, degloc, sg0, sg1, si0, si1):
    c = lax.axis_index("c")
    s = lax.axis_index("s")
    zvec = jnp.zeros((L,), dtype=jnp.float32)
    ones = jnp.ones((L,), dtype=jnp.float32)
    idx_b = (idx0, idx1)
    dst_b = (dst0, dst1)
    rows_b = (rows0, rows1)
    sg_b = (sg0, sg1)
    si_b = (si0, si1)

    def zrow(r, carry):
        for j in range(H // L):
            rows0[r, pl.ds(j * L, L)] = zvec
        return carry

    lax.fori_loop(0, RC, zrow, 0)

    def zdeg(i, carry):
        degloc[pl.ds(i * L, L)] = zvec
        return carry

    lax.fori_loop(0, N // L, zdeg, 0)

    for k in range(RPT):
        cid = k * NS + s

        @pl.when(cid < NRCH)
        def _():
            pltpu.sync_copy(rows0, acc.at[pl.ds(cid * RC, RC)])

    plsc.subcore_barrier()

    base = s * EPT

    sbase = c * E + base

    def start_load_idx(k, b):
        eoff = k * EC
        pltpu.async_copy(src2.at[pl.ds(sbase + eoff, EC)], idx_b[b], si_b[b])
        pltpu.async_copy(dst.at[pl.ds(base + eoff, EC)], dst_b[b], si_b[b])

    def wait_load_idx(k, b):
        eoff = k * EC
        pltpu.make_async_copy(src2.at[pl.ds(sbase + eoff, EC)], idx_b[b],
                              si_b[b]).wait()
        pltpu.make_async_copy(dst.at[pl.ds(base + eoff, EC)], dst_b[b],
                              si_b[b]).wait()

    def start_gather(b):
        pltpu.async_copy(xview.at[idx_b[b]], rows_b[b], sg_b[b])

    def wait_gather(b):
        pltpu.make_async_copy(xview.at[idx_b[b]], rows_b[b], sg_b[b]).wait()

    start_load_idx(0, 0)
    wait_load_idx(0, 0)
    start_gather(0)
    start_load_idx(1, 1)

    def body(k2, carry):
        for b in range(2):
            k = k2 * 2 + b

            @pl.when(k < NCH)
            def _():
                wait_gather(b)

                @pl.when(k + 1 < NCH)
                def _():
                    wait_load_idx(k + 1, 1 - b)
                    start_gather(1 - b)

                pltpu.sync_copy(rows_b[b], acc.at[dst_b[b]], add=True)

                @pl.when(c == 0)
                def _():
                    for j in range(EC // L):
                        iv = dst_b[b][pl.ds(j * L, L)]
                        plsc.addupdate_scatter(degloc, [iv], ones)

                @pl.when(k + 2 < NCH)
                def _():
                    start_load_idx(k + 2, b)

        return carry

    lax.fori_loop(0, (NCH + 1) // 2, body, 0)

    @pl.when(c == 0)
    def _():
        pltpu.sync_copy(degloc, dego.at[pl.ds(s * NP, N)])

    plsc.subcore_barrier()

    for k in range(RPT):
        cid = k * NS + s

        @pl.when(cid < NRCH)
        def _():
            row0 = cid * RC

            @pl.when(c == 0)
            def _():
                pltpu.sync_copy(acc.at[pl.ds(row0, RC)],
                                h0o.at[pl.ds(row0, RC)])

            @pl.when(c == 1)
            def _():
                pltpu.sync_copy(acc.at[pl.ds(row0, RC)],
                                h1o.at[pl.ds(row0, RC)])


_sc_agg = pl.kernel(
    _sc_body,
    out_type=(
        jax.ShapeDtypeStruct((N, H), jnp.float32),
        jax.ShapeDtypeStruct((N, H), jnp.float32),
        jax.ShapeDtypeStruct((NS * NP,), jnp.float32),
    ),
    mesh=plsc.VectorSubcoreMesh(core_axis_name="c", subcore_axis_name="s"),
    compiler_params=pltpu.CompilerParams(needs_layout_passes=False),
    scratch_types=[
        pltpu.VMEM_SHARED((N, H), jnp.float32),
        pltpu.VMEM((EC,), jnp.int32),
        pltpu.VMEM((EC,), jnp.int32),
        pltpu.VMEM((EC,), jnp.int32),
        pltpu.VMEM((EC,), jnp.int32),
        pltpu.VMEM((EC, H), jnp.float32),
        pltpu.VMEM((EC, H), jnp.float32),
        pltpu.VMEM((N,), jnp.float32),
        pltpu.SemaphoreType.DMA,
        pltpu.SemaphoreType.DMA,
        pltpu.SemaphoreType.DMA,
        pltpu.SemaphoreType.DMA,
    ],
)


BN = 2000


def _tc_body(x_ref, h0_ref, h1_ref, dg_ref, ws_ref, wn0_ref, wn1_ref,
             b_ref, o_ref):
    deg = jnp.sum(dg_ref[...], axis=1)
    rdeg = (1.0 / jnp.maximum(deg, 1.0))[:, None]
    o_ref[...] = (
        jnp.dot(x_ref[...], ws_ref[...], preferred_element_type=jnp.float32)
        + jnp.dot(h0_ref[...] * rdeg, wn0_ref[...],
                  preferred_element_type=jnp.float32)
        + jnp.dot(h1_ref[...] * rdeg, wn1_ref[...],
                  preferred_element_type=jnp.float32)
        + b_ref[...]
    )


_tc_dense = pl.pallas_call(
    _tc_body,
    grid=(N // BN,),
    in_specs=[
        pl.BlockSpec((BN, D), lambda i: (i, 0)),
        pl.BlockSpec((BN, H), lambda i: (i, 0)),
        pl.BlockSpec((BN, H), lambda i: (i, 0)),
        pl.BlockSpec((BN, NS), lambda i: (i, 0)),
        pl.BlockSpec((D, D), lambda i: (0, 0)),
        pl.BlockSpec((H, D), lambda i: (0, 0)),
        pl.BlockSpec((H, D), lambda i: (0, 0)),
        pl.BlockSpec((1, D), lambda i: (0, 0)),
    ],
    out_specs=pl.BlockSpec((BN, D), lambda i: (i, 0)),
    out_shape=jax.ShapeDtypeStruct((N, D), jnp.float32),
)


def kernel(x, edge_index, W_self, W_neigh, b):
    src = edge_index[0].astype(jnp.int32)
    dst = edge_index[1].astype(jnp.int32)
    xview = x.reshape(2 * N, H)
    s2 = src * 2
    src2 = jnp.concatenate([s2, s2 + 1])
    h0, h1, dego = _sc_agg(xview, src2, dst)
    deg16 = dego.reshape(NS, NP)[:, :N].T
    return _tc_dense(x, h0, h1, deg16, W_self, W_neigh[:H], W_neigh[H:],
                     b.reshape(1, D))
